# baseline (speedup 1.0000x reference)
import functools

import jax
import jax.numpy as jnp
from jax import lax
from jax.experimental import pallas as pl
from jax.experimental.pallas import tpu as pltpu
from jax.experimental.pallas import tpu_sc as plsc

N_TAG = 1000
N_VID = 16384
D = 768
E = 50000
FLAT = N_TAG * N_VID

NC = 2
NS = 16
L = 16

CH = 3136
NVREG = CH // L
ROWS_PER_SC = N_TAG // NC
R = 50
NPASS = ROWS_PER_SC // R
SLICE = R * N_VID
SHARE = SLICE // NS
SINK_PAD = 128
HUGE = 0x7FFFFFF0

_mesh = plsc.VectorSubcoreMesh(
    core_axis_name="c", subcore_axis_name="s", num_cores=NC, num_subcores=NS
)


@functools.partial(
    pl.kernel,
    out_type=(
        jax.ShapeDtypeStruct((FLAT,), jnp.float32),
        jax.ShapeDtypeStruct((FLAT,), jnp.float32),
    ),
    mesh=_mesh,
    scratch_types=[
        pltpu.VMEM((CH,), jnp.int32),
        pltpu.VMEM((CH,), jnp.int32),
        pltpu.VMEM((CH,), jnp.int32),
        pltpu.VMEM((CH,), jnp.int32),
        pltpu.VMEM((CH,), jnp.int32),
        pltpu.VMEM((CH,), jnp.int32),
        pltpu.VMEM((CH,), jnp.float32),
        pltpu.VMEM((SHARE,), jnp.float32),
        pltpu.VMEM_SHARED((SLICE + SINK_PAD,), jnp.float32),
        pltpu.SemaphoreType.DMA,
        pltpu.SemaphoreType.DMA,
    ],
)
def _sc_counts(pt, pv, nt, nv, out_all, out_pos,
               tbuf, vbuf, lpos, lneg, idxp, idxn, ones, zeros,
               acc, sem0, sem1):
    c = lax.axis_index("c")
    s = lax.axis_index("s")
    base = jnp.minimum(s * CH, E - CH)
    lane = lax.iota(jnp.int32, L)

    def build(t_hbm, v_hbm, lbuf):
        pltpu.sync_copy(t_hbm.at[pl.ds(base, CH)], tbuf)
        pltpu.sync_copy(v_hbm.at[pl.ds(base, CH)], vbuf)

        def body(j, _):
            t = tbuf[pl.ds(j * L, L)]
            v = vbuf[pl.ds(j * L, L)]
            flat = t * N_VID + v
            gpos = base + j * L + lane
            mine = gpos >= s * CH
            lbuf[pl.ds(j * L, L)] = jnp.where(mine, flat, HUGE)
            return 0

        lax.fori_loop(0, NVREG, body, 0)

    build(pt, pv, lpos)
    build(nt, nv, lneg)

    def fill(j, _):
        ones[pl.ds(j * L, L)] = jnp.full((L,), 1.0, jnp.float32)
        return 0

    lax.fori_loop(0, NVREG, fill, 0)

    def fillz(j, _):
        zeros[pl.ds(j * L, L)] = jnp.zeros((L,), jnp.float32)
        return 0

    lax.fori_loop(0, SHARE // L, fillz, 0)

    pltpu.sync_copy(zeros, acc.at[pl.ds(s * SHARE, SHARE)])
    plsc.subcore_barrier()

    def mk_idx(lbuf, ibuf, lo):
        def body(j, _):
            u = lbuf[pl.ds(j * L, L)] - lo
            inr = (u >= 0) & (u < SLICE)
            sink = SLICE + ((j & 7) * L) + lane
            ibuf[pl.ds(j * L, L)] = jnp.where(inr, u, sink)
            return 0

        lax.fori_loop(0, NVREG, body, 0)

    def phase(lbufs, ibufs, out):
        def do_pass(p, _):
            lo = (c * ROWS_PER_SC + p * R) * N_VID
            for lbuf, ibuf in zip(lbufs, ibufs):
                mk_idx(lbuf, ibuf, lo)

            cps = [
                pltpu.async_copy(ones, acc.at[ibuf], sem, add=True)
                for ibuf, sem in zip(ibufs, (sem0, sem1))
            ]
            for cp in cps:
                cp.wait()
            plsc.subcore_barrier()

            off = s * SHARE
            pltpu.sync_copy(acc.at[pl.ds(off, SHARE)],
                            out.at[pl.ds(lo + off, SHARE)])
            pltpu.sync_copy(zeros, acc.at[pl.ds(off, SHARE)])
            plsc.subcore_barrier()
            return 0

        lax.fori_loop(0, NPASS, do_pass, 0)

    phase([lpos, lneg], [idxp, idxn], out_all)
    phase([lpos], [idxp], out_pos)


BT = 200
BV = 2048


def _tc_body(a_ref, b_ref, cnt_ref, o_ref):
    s = lax.dot_general(
        a_ref[...], b_ref[...],
        dimension_numbers=(((1,), (1,)), ((), ())),
        preferred_element_type=jnp.float32,
    )
    o_ref[...] = s * cnt_ref[...]


def _tc_score(h_tag, h_vid, counts):
    return pl.pallas_call(
        _tc_body,
        grid=(N_TAG // BT, N_VID // BV),
        in_specs=[
            pl.BlockSpec((BT, D), lambda i, j: (i, 0)),
            pl.BlockSpec((BV, D), lambda i, j: (j, 0)),
            pl.BlockSpec((BT, BV), lambda i, j: (i, j)),
        ],
        out_specs=pl.BlockSpec((BT, BV), lambda i, j: (i, j)),
        out_shape=jax.ShapeDtypeStruct((N_TAG, N_VID), jnp.float32),
    )(h_tag, h_vid, counts)


def kernel(h_tag, h_vid, pos_tag_idx, pos_vid_idx, neg_tag_idx, neg_vid_idx):
    counts_all, counts_pos = _sc_counts(
        pos_tag_idx, pos_vid_idx, neg_tag_idx, neg_vid_idx
    )
    counts_all = counts_all.reshape(N_TAG, N_VID)
    labels = counts_pos.reshape(N_TAG, N_VID)
    cls_score = _tc_score(h_tag, h_vid, counts_all)
    return (cls_score, labels)


# speedup vs baseline: 1.0174x; 1.0174x over previous
import functools

import jax
import jax.numpy as jnp
from jax import lax
from jax.experimental import pallas as pl
from jax.experimental.pallas import tpu as pltpu
from jax.experimental.pallas import tpu_sc as plsc

N_TAG = 1000
N_VID = 16384
D = 768
E = 50000
FLAT = N_TAG * N_VID

NC = 2
NS = 16
L = 16

CH = 3136
NVREG = CH // L
ROWS_PER_SC = N_TAG // NC
R = 50
NPASS = ROWS_PER_SC // R
SLICE = R * N_VID
SHARE = SLICE // NS
SINK_PAD = 128
HUGE = 0x7FFFFFF0

_mesh = plsc.VectorSubcoreMesh(
    core_axis_name="c", subcore_axis_name="s", num_cores=NC, num_subcores=NS
)


@functools.partial(
    pl.kernel,
    out_type=(
        jax.ShapeDtypeStruct((FLAT,), jnp.float32),
        jax.ShapeDtypeStruct((FLAT,), jnp.float32),
    ),
    mesh=_mesh,
    scratch_types=[
        pltpu.VMEM((CH,), jnp.int32),
        pltpu.VMEM((CH,), jnp.int32),
        pltpu.VMEM((CH,), jnp.int32),
        pltpu.VMEM((CH,), jnp.int32),
        pltpu.VMEM((CH,), jnp.int32),
        pltpu.VMEM((CH,), jnp.int32),
        pltpu.VMEM((CH,), jnp.float32),
        pltpu.VMEM((SHARE,), jnp.float32),
        pltpu.VMEM_SHARED((SLICE + SINK_PAD,), jnp.float32),
        pltpu.SemaphoreType.DMA,
        pltpu.SemaphoreType.DMA,
    ],
)
def _sc_counts(pt, pv, nt, nv, out_all, out_pos,
               tbuf, vbuf, lpos, lneg, idxp, idxn, ones, zeros,
               acc, sem0, sem1):
    c = lax.axis_index("c")
    s = lax.axis_index("s")
    base = jnp.minimum(s * CH, E - CH)
    lane = lax.iota(jnp.int32, L)

    def build(t_hbm, v_hbm, lbuf):
        pltpu.sync_copy(t_hbm.at[pl.ds(base, CH)], tbuf)
        pltpu.sync_copy(v_hbm.at[pl.ds(base, CH)], vbuf)

        def body(j, _):
            t = tbuf[pl.ds(j * L, L)]
            v = vbuf[pl.ds(j * L, L)]
            flat = t * N_VID + v
            gpos = base + j * L + lane
            mine = gpos >= s * CH
            lbuf[pl.ds(j * L, L)] = jnp.where(mine, flat, HUGE)
            return 0

        lax.fori_loop(0, NVREG, body, 0)

    build(pt, pv, lpos)
    build(nt, nv, lneg)

    def fill(j, _):
        ones[pl.ds(j * L, L)] = jnp.full((L,), 1.0, jnp.float32)
        return 0

    lax.fori_loop(0, NVREG, fill, 0)

    def fillz(j, _):
        zeros[pl.ds(j * L, L)] = jnp.zeros((L,), jnp.float32)
        return 0

    lax.fori_loop(0, SHARE // L, fillz, 0)

    pltpu.sync_copy(zeros, acc.at[pl.ds(s * SHARE, SHARE)])
    plsc.subcore_barrier()

    def mk_idx(lbuf, ibuf, lo):
        def body(j, _):
            u = lbuf[pl.ds(j * L, L)] - lo
            inr = (u >= 0) & (u < SLICE)
            sink = SLICE + ((j & 7) * L) + lane
            ibuf[pl.ds(j * L, L)] = jnp.where(inr, u, sink)
            return 0

        lax.fori_loop(0, NVREG, body, 0)

    def phase(lbufs, ibufs, out):
        def do_pass(p, _):
            lo = (c * ROWS_PER_SC + p * R) * N_VID
            for lbuf, ibuf in zip(lbufs, ibufs):
                mk_idx(lbuf, ibuf, lo)

            cps = [
                pltpu.async_copy(ones, acc.at[ibuf], sem, add=True)
                for ibuf, sem in zip(ibufs, (sem0, sem1))
            ]
            for cp in cps:
                cp.wait()
            plsc.subcore_barrier()

            off = s * SHARE
            pltpu.sync_copy(acc.at[pl.ds(off, SHARE)],
                            out.at[pl.ds(lo + off, SHARE)])
            plsc.subcore_barrier()

            zps = [
                pltpu.async_copy(zeros.at[pl.ds(0, CH)], acc.at[ibuf], sem)
                for ibuf, sem in zip(ibufs, (sem0, sem1))
            ]
            for zp in zps:
                zp.wait()
            plsc.subcore_barrier()
            return 0

        lax.fori_loop(0, NPASS, do_pass, 0)

    phase([lpos, lneg], [idxp, idxn], out_all)
    phase([lpos], [idxp], out_pos)


BT = 200
BV = 2048


def _tc_body(a_ref, b_ref, cnt_ref, o_ref):
    s = lax.dot_general(
        a_ref[...], b_ref[...],
        dimension_numbers=(((1,), (1,)), ((), ())),
        preferred_element_type=jnp.float32,
    )
    o_ref[...] = s * cnt_ref[...]


def _tc_score(h_tag, h_vid, counts):
    return pl.pallas_call(
        _tc_body,
        grid=(N_TAG // BT, N_VID // BV),
        in_specs=[
            pl.BlockSpec((BT, D), lambda i, j: (i, 0)),
            pl.BlockSpec((BV, D), lambda i, j: (j, 0)),
            pl.BlockSpec((BT, BV), lambda i, j: (i, j)),
        ],
        out_specs=pl.BlockSpec((BT, BV), lambda i, j: (i, j)),
        out_shape=jax.ShapeDtypeStruct((N_TAG, N_VID), jnp.float32),
    )(h_tag, h_vid, counts)


def kernel(h_tag, h_vid, pos_tag_idx, pos_vid_idx, neg_tag_idx, neg_vid_idx):
    counts_all, counts_pos = _sc_counts(
        pos_tag_idx, pos_vid_idx, neg_tag_idx, neg_vid_idx
    )
    counts_all = counts_all.reshape(N_TAG, N_VID)
    labels = counts_pos.reshape(N_TAG, N_VID)
    cls_score = _tc_score(h_tag, h_vid, counts_all)
    return (cls_score, labels)


# speedup vs baseline: 1.0748x; 1.0563x over previous
import functools

import jax
import jax.numpy as jnp
from jax import lax
from jax.experimental import pallas as pl
from jax.experimental.pallas import tpu as pltpu
from jax.experimental.pallas import tpu_sc as plsc

N_TAG = 1000
N_VID = 16384
D = 768
E = 50000
FLAT = N_TAG * N_VID

NC = 2
NS = 16
L = 16

CH = 3136
NVREG = CH // L
ROWS_PER_SC = N_TAG // NC
R = 80
NPASS = -(-ROWS_PER_SC // R)
SLICE = R * N_VID
SHARE = SLICE // NS
SINK_PAD = 128
HUGE = 0x7FFFFFF0

_mesh = plsc.VectorSubcoreMesh(
    core_axis_name="c", subcore_axis_name="s", num_cores=NC, num_subcores=NS
)


@functools.partial(
    pl.kernel,
    out_type=(
        jax.ShapeDtypeStruct((FLAT,), jnp.float32),
        jax.ShapeDtypeStruct((FLAT,), jnp.float32),
    ),
    mesh=_mesh,
    scratch_types=[
        pltpu.VMEM((CH,), jnp.int32),
        pltpu.VMEM((CH,), jnp.int32),
        pltpu.VMEM((CH,), jnp.int32),
        pltpu.VMEM((CH,), jnp.int32),
        pltpu.VMEM((CH,), jnp.int32),
        pltpu.VMEM((CH,), jnp.int32),
        pltpu.VMEM((CH,), jnp.float32),
        pltpu.VMEM((SHARE // 4,), jnp.float32),
        pltpu.VMEM_SHARED((SLICE + SINK_PAD,), jnp.float32),
        pltpu.SemaphoreType.DMA,
        pltpu.SemaphoreType.DMA,
    ],
)
def _sc_counts(pt, pv, nt, nv, out_all, out_pos,
               tbuf, vbuf, lpos, lneg, idxp, idxn, ones, zeros,
               acc, sem0, sem1):
    c = lax.axis_index("c")
    s = lax.axis_index("s")
    base = jnp.minimum(s * CH, E - CH)
    lane = lax.iota(jnp.int32, L)

    def build(t_hbm, v_hbm, lbuf):
        pltpu.sync_copy(t_hbm.at[pl.ds(base, CH)], tbuf)
        pltpu.sync_copy(v_hbm.at[pl.ds(base, CH)], vbuf)

        def body(j, _):
            t = tbuf[pl.ds(j * L, L)]
            v = vbuf[pl.ds(j * L, L)]
            flat = t * N_VID + v
            gpos = base + j * L + lane
            mine = gpos >= s * CH
            lbuf[pl.ds(j * L, L)] = jnp.where(mine, flat, HUGE)
            return 0

        lax.fori_loop(0, NVREG, body, 0)

    build(pt, pv, lpos)
    build(nt, nv, lneg)

    def fill(j, _):
        ones[pl.ds(j * L, L)] = jnp.full((L,), 1.0, jnp.float32)
        return 0

    lax.fori_loop(0, NVREG, fill, 0)

    def fillz(j, _):
        zeros[pl.ds(j * L, L)] = jnp.zeros((L,), jnp.float32)
        return 0

    lax.fori_loop(0, SHARE // 4 // L, fillz, 0)

    def zinit(k, _):
        pltpu.sync_copy(zeros, acc.at[pl.ds(s * SHARE + k * (SHARE // 4),
                                            SHARE // 4)])
        return 0

    lax.fori_loop(0, 4, zinit, 0)
    plsc.subcore_barrier()

    def mk_idx(lbuf, ibuf, lo):
        def body(j, _):
            u = lbuf[pl.ds(j * L, L)] - lo
            inr = (u >= 0) & (u < SLICE)
            sink = SLICE + ((j & 7) * L) + lane
            ibuf[pl.ds(j * L, L)] = jnp.where(inr, u, sink)
            return 0

        lax.fori_loop(0, NVREG, body, 0)

    def phase(lbufs, ibufs, out):
        def do_pass(p, _):
            lo = (c * ROWS_PER_SC
                  + jnp.minimum(p * R, ROWS_PER_SC - R)) * N_VID
            for lbuf, ibuf in zip(lbufs, ibufs):
                mk_idx(lbuf, ibuf, lo)
                pltpu.sync_copy(ones, acc.at[ibuf], add=True)
            plsc.subcore_barrier()

            off = s * SHARE
            pltpu.sync_copy(acc.at[pl.ds(off, SHARE)],
                            out.at[pl.ds(lo + off, SHARE)])
            plsc.subcore_barrier()

            for ibuf in ibufs:
                pltpu.sync_copy(zeros.at[pl.ds(0, CH)], acc.at[ibuf])
            plsc.subcore_barrier()
            return 0

        lax.fori_loop(0, NPASS, do_pass, 0)

    phase([lpos, lneg], [idxp, idxn], out_all)
    phase([lpos], [idxp], out_pos)


BT = 200
BV = 2048


def _tc_body(a_ref, b_ref, cnt_ref, o_ref):
    s = lax.dot_general(
        a_ref[...], b_ref[...],
        dimension_numbers=(((1,), (1,)), ((), ())),
        preferred_element_type=jnp.float32,
    )
    o_ref[...] = s * cnt_ref[...]


def _tc_score(h_tag, h_vid, counts):
    return pl.pallas_call(
        _tc_body,
        grid=(N_TAG // BT, N_VID // BV),
        in_specs=[
            pl.BlockSpec((BT, D), lambda i, j: (i, 0)),
            pl.BlockSpec((BV, D), lambda i, j: (j, 0)),
            pl.BlockSpec((BT, BV), lambda i, j: (i, j)),
        ],
        out_specs=pl.BlockSpec((BT, BV), lambda i, j: (i, j)),
        out_shape=jax.ShapeDtypeStruct((N_TAG, N_VID), jnp.float32),
    )(h_tag, h_vid, counts)


def kernel(h_tag, h_vid, pos_tag_idx, pos_vid_idx, neg_tag_idx, neg_vid_idx):
    counts_all, counts_pos = _sc_counts(
        pos_tag_idx, pos_vid_idx, neg_tag_idx, neg_vid_idx
    )
    counts_all = counts_all.reshape(N_TAG, N_VID)
    labels = counts_pos.reshape(N_TAG, N_VID)
    cls_score = _tc_score(h_tag, h_vid, counts_all)
    return (cls_score, labels)


# speedup vs baseline: 1.1279x; 1.0495x over previous
import functools

import jax
import jax.numpy as jnp
from jax import lax
from jax.experimental import pallas as pl
from jax.experimental.pallas import tpu as pltpu
from jax.experimental.pallas import tpu_sc as plsc

N_TAG = 1000
N_VID = 16384
D = 768
E = 50000
FLAT = N_TAG * N_VID

NC = 2
NS = 16
L = 16

CH = 3136
NVREG = CH // L
ROWS_PER_SC = N_TAG // NC
R = 80
NPASS = -(-ROWS_PER_SC // R)
SLICE = R * N_VID
SHARE = SLICE // NS
SINK_PAD = 128
HUGE = 0x7FFFFFF0

_mesh = plsc.VectorSubcoreMesh(
    core_axis_name="c", subcore_axis_name="s", num_cores=NC, num_subcores=NS
)


@functools.partial(
    pl.kernel,
    out_type=(
        jax.ShapeDtypeStruct((FLAT,), jnp.float32),
        jax.ShapeDtypeStruct((FLAT,), jnp.float32),
    ),
    mesh=_mesh,
    scratch_types=[
        pltpu.VMEM((CH,), jnp.int32),
        pltpu.VMEM((CH,), jnp.int32),
        pltpu.VMEM((CH,), jnp.int32),
        pltpu.VMEM((CH,), jnp.int32),
        pltpu.VMEM((CH,), jnp.int32),
        pltpu.VMEM((CH,), jnp.int32),
        pltpu.VMEM((CH,), jnp.float32),
        pltpu.VMEM((SHARE // 4,), jnp.float32),
        pltpu.VMEM_SHARED((SLICE + SINK_PAD,), jnp.float32),
        pltpu.SemaphoreType.DMA,
        pltpu.SemaphoreType.DMA,
    ],
)
def _sc_counts(pt, pv, nt, nv, out_all, out_pos,
               tbuf, vbuf, lpos, lneg, idxp, idxn, ones, zeros,
               acc, sem0, sem1):
    c = lax.axis_index("c")
    s = lax.axis_index("s")
    base = jnp.minimum(s * CH, E - CH)
    lane = lax.iota(jnp.int32, L)

    def build(t_hbm, v_hbm, lbuf):
        pltpu.sync_copy(t_hbm.at[pl.ds(base, CH)], tbuf)
        pltpu.sync_copy(v_hbm.at[pl.ds(base, CH)], vbuf)

        def body(j, _):
            t = tbuf[pl.ds(j * L, L)]
            v = vbuf[pl.ds(j * L, L)]
            flat = t * N_VID + v
            gpos = base + j * L + lane
            mine = gpos >= s * CH
            lbuf[pl.ds(j * L, L)] = jnp.where(mine, flat, HUGE)
            return 0

        lax.fori_loop(0, NVREG, body, 0)

    build(pt, pv, lpos)
    build(nt, nv, lneg)

    def fill(j, _):
        ones[pl.ds(j * L, L)] = jnp.full((L,), 1.0, jnp.float32)
        return 0

    lax.fori_loop(0, NVREG, fill, 0)

    def fillz(j, _):
        zeros[pl.ds(j * L, L)] = jnp.zeros((L,), jnp.float32)
        return 0

    lax.fori_loop(0, SHARE // 4 // L, fillz, 0)

    def zinit(k, _):
        pltpu.sync_copy(zeros, acc.at[pl.ds(s * SHARE + k * (SHARE // 4),
                                            SHARE // 4)])
        return 0

    lax.fori_loop(0, 4, zinit, 0)
    plsc.subcore_barrier()

    def mk_idx(lbuf, ibuf, lo):
        def body(j, _):
            u = lbuf[pl.ds(j * L, L)] - lo
            inr = (u >= 0) & (u < SLICE)
            sink = SLICE + ((j & 7) * L) + lane
            ibuf[pl.ds(j * L, L)] = jnp.where(inr, u, sink)
            return 0

        lax.fori_loop(0, NVREG, body, 0)

    def phase(lbufs, ibufs, out):
        def do_pass(p, _):
            lo = (c * ROWS_PER_SC
                  + jnp.minimum(p * R, ROWS_PER_SC - R)) * N_VID
            for lbuf, ibuf in zip(lbufs, ibufs):
                mk_idx(lbuf, ibuf, lo)
                pltpu.sync_copy(ones, acc.at[ibuf], add=True)
            plsc.subcore_barrier()

            off = s * SHARE
            pltpu.sync_copy(acc.at[pl.ds(off, SHARE)],
                            out.at[pl.ds(lo + off, SHARE)])
            plsc.subcore_barrier()

            for ibuf in ibufs:
                pltpu.sync_copy(zeros.at[pl.ds(0, CH)], acc.at[ibuf])
            plsc.subcore_barrier()
            return 0

        lax.fori_loop(0, NPASS, do_pass, 0)

    phase([lpos, lneg], [idxp, idxn], out_all)
    phase([lpos], [idxp], out_pos)


BT = 200
BV = 2048


def _tc_body(a_ref, b_ref, cnt_ref, o_ref):
    s = lax.dot_general(
        a_ref[...], b_ref[...],
        dimension_numbers=(((1,), (1,)), ((), ())),
        preferred_element_type=jnp.float32,
    )
    o_ref[...] = s * cnt_ref[...]


def _tc_score(h_tag, h_vid, counts):
    return pl.pallas_call(
        _tc_body,
        grid=(N_TAG // BT, N_VID // BV),
        in_specs=[
            pl.BlockSpec((BT, D), lambda i, j: (i, 0)),
            pl.BlockSpec((BV, D), lambda i, j: (j, 0)),
            pl.BlockSpec((BT, BV), lambda i, j: (i, j)),
        ],
        out_specs=pl.BlockSpec((BT, BV), lambda i, j: (i, j)),
        out_shape=jax.ShapeDtypeStruct((N_TAG, N_VID), jnp.float32),
    )(h_tag, h_vid, counts)


def kernel(h_tag, h_vid, pos_tag_idx, pos_vid_idx, neg_tag_idx, neg_vid_idx):
    counts_all, counts_pos = _sc_counts(
        pos_tag_idx, pos_vid_idx, neg_tag_idx, neg_vid_idx
    )
    counts_all = counts_all.reshape(N_TAG, N_VID)
    labels = counts_pos.reshape(N_TAG, N_VID)
    cls_score = _tc_score(h_tag.astype(jnp.bfloat16),
                          h_vid.astype(jnp.bfloat16), counts_all)
    return (cls_score, labels)


# speedup vs baseline: 1.3063x; 1.1582x over previous
import functools

import jax
import jax.numpy as jnp
from jax import lax
from jax.experimental import pallas as pl
from jax.experimental.pallas import tpu as pltpu
from jax.experimental.pallas import tpu_sc as plsc

N_TAG = 1000
N_VID = 16384
D = 768
E = 50000
FLAT = N_TAG * N_VID

NC = 2
NS = 16
L = 16

CH = 3136
NVREG = CH // L
ROWS_PER_SC = N_TAG // NC
R = 80
NPASS = -(-ROWS_PER_SC // R)
SLICE = R * N_VID
SHARE = SLICE // NS
SINK_PAD = 128
HUGE = 0x7FFFFFF0

_mesh = plsc.VectorSubcoreMesh(
    core_axis_name="c", subcore_axis_name="s", num_cores=NC, num_subcores=NS
)


def _make_sc_counts(nlists):
    @functools.partial(
        pl.kernel,
        out_type=jax.ShapeDtypeStruct((FLAT,), jnp.float32),
        mesh=_mesh,
        scratch_types=[
            pltpu.VMEM((CH,), jnp.int32),
            pltpu.VMEM((CH,), jnp.int32),
        ] + [pltpu.VMEM((CH,), jnp.int32)] * nlists
          + [pltpu.VMEM((CH,), jnp.int32)] * nlists
          + [
            pltpu.VMEM((CH,), jnp.float32),
            pltpu.VMEM((SHARE // 4,), jnp.float32),
            pltpu.VMEM_SHARED((SLICE + SINK_PAD,), jnp.float32),
        ],
    )
    def sc_counts(*args):
        idx_hbm = args[:2 * nlists]
        out = args[2 * nlists]
        tbuf, vbuf = args[2 * nlists + 1:2 * nlists + 3]
        lbufs = args[2 * nlists + 3:2 * nlists + 3 + nlists]
        ibufs = args[2 * nlists + 3 + nlists:2 * nlists + 3 + 2 * nlists]
        ones, zeros, acc = args[2 * nlists + 3 + 2 * nlists:]

        c = lax.axis_index("c")
        s = lax.axis_index("s")
        base = jnp.minimum(s * CH, E - CH)
        lane = lax.iota(jnp.int32, L)

        def build(t_hbm, v_hbm, lbuf):
            pltpu.sync_copy(t_hbm.at[pl.ds(base, CH)], tbuf)
            pltpu.sync_copy(v_hbm.at[pl.ds(base, CH)], vbuf)

            def body(j, _):
                t = tbuf[pl.ds(j * L, L)]
                v = vbuf[pl.ds(j * L, L)]
                flat = t * N_VID + v
                gpos = base + j * L + lane
                mine = gpos >= s * CH
                lbuf[pl.ds(j * L, L)] = jnp.where(mine, flat, HUGE)
                return 0

            lax.fori_loop(0, NVREG, body, 0)

        for i in range(nlists):
            build(idx_hbm[2 * i], idx_hbm[2 * i + 1], lbufs[i])

        def fill(j, _):
            ones[pl.ds(j * L, L)] = jnp.full((L,), 1.0, jnp.float32)
            return 0

        lax.fori_loop(0, NVREG, fill, 0)

        def fillz(j, _):
            zeros[pl.ds(j * L, L)] = jnp.zeros((L,), jnp.float32)
            return 0

        lax.fori_loop(0, SHARE // 4 // L, fillz, 0)

        def zinit(k, _):
            pltpu.sync_copy(zeros, acc.at[pl.ds(s * SHARE + k * (SHARE // 4),
                                                SHARE // 4)])
            return 0

        lax.fori_loop(0, 4, zinit, 0)
        plsc.subcore_barrier()

        def mk_idx(lbuf, ibuf, lo):
            def body(j, _):
                u = lbuf[pl.ds(j * L, L)] - lo
                inr = (u >= 0) & (u < SLICE)
                sink = SLICE + ((j & 7) * L) + lane
                ibuf[pl.ds(j * L, L)] = jnp.where(inr, u, sink)
                return 0

            lax.fori_loop(0, NVREG, body, 0)

        def do_pass(p, _):
            lo = (c * ROWS_PER_SC
                  + jnp.minimum(p * R, ROWS_PER_SC - R)) * N_VID
            for lbuf, ibuf in zip(lbufs, ibufs):
                mk_idx(lbuf, ibuf, lo)
                pltpu.sync_copy(ones, acc.at[ibuf], add=True)
            plsc.subcore_barrier()

            off = s * SHARE
            pltpu.sync_copy(acc.at[pl.ds(off, SHARE)],
                            out.at[pl.ds(lo + off, SHARE)])
            plsc.subcore_barrier()

            for ibuf in ibufs:
                pltpu.sync_copy(zeros.at[pl.ds(0, CH)], acc.at[ibuf])
            plsc.subcore_barrier()
            return 0

        lax.fori_loop(0, NPASS, do_pass, 0)

    return sc_counts


_sc_counts_all = _make_sc_counts(2)
_sc_counts_pos = _make_sc_counts(1)


BT = 200
BV = 2048


def _tc_body(a_ref, b_ref, cnt_ref, o_ref):
    s = lax.dot_general(
        a_ref[...], b_ref[...],
        dimension_numbers=(((1,), (1,)), ((), ())),
        preferred_element_type=jnp.float32,
    )
    o_ref[...] = s * cnt_ref[...]


def _tc_score(h_tag, h_vid, counts):
    return pl.pallas_call(
        _tc_body,
        grid=(N_TAG // BT, N_VID // BV),
        in_specs=[
            pl.BlockSpec((BT, D), lambda i, j: (i, 0)),
            pl.BlockSpec((BV, D), lambda i, j: (j, 0)),
            pl.BlockSpec((BT, BV), lambda i, j: (i, j)),
        ],
        out_specs=pl.BlockSpec((BT, BV), lambda i, j: (i, j)),
        out_shape=jax.ShapeDtypeStruct((N_TAG, N_VID), jnp.float32),
    )(h_tag, h_vid, counts)


def kernel(h_tag, h_vid, pos_tag_idx, pos_vid_idx, neg_tag_idx, neg_vid_idx):
    counts_all = _sc_counts_all(
        pos_tag_idx, pos_vid_idx, neg_tag_idx, neg_vid_idx
    ).reshape(N_TAG, N_VID)
    labels = _sc_counts_pos(pos_tag_idx, pos_vid_idx).reshape(N_TAG, N_VID)
    cls_score = _tc_score(h_tag.astype(jnp.bfloat16),
                          h_vid.astype(jnp.bfloat16), counts_all)
    return (cls_score, labels)


# speedup vs baseline: 1.3529x; 1.0356x over previous
import functools

import jax
import jax.numpy as jnp
from jax import lax
from jax.experimental import pallas as pl
from jax.experimental.pallas import tpu as pltpu
from jax.experimental.pallas import tpu_sc as plsc

N_TAG = 1000
N_VID = 16384
D = 768
E = 50000
FLAT = N_TAG * N_VID

NC = 2
NS = 16
L = 16

CH = 3136
NVREG = CH // L
ROWS_PER_SC = N_TAG // NC
R = 79
NPASS = -(-ROWS_PER_SC // R)
SLICE = R * N_VID
SHARE = SLICE // NS
SINK_PAD = 128
HUGE = 0x7FFFFFF0

_mesh = plsc.VectorSubcoreMesh(
    core_axis_name="c", subcore_axis_name="s", num_cores=NC, num_subcores=NS
)


def _make_sc_counts(nlists):
    @functools.partial(
        pl.kernel,
        out_type=jax.ShapeDtypeStruct((FLAT,), jnp.float32),
        mesh=_mesh,
        scratch_types=[
            pltpu.VMEM((CH,), jnp.int32),
            pltpu.VMEM((CH,), jnp.int32),
        ] + [pltpu.VMEM((CH,), jnp.int32)] * nlists
          + [pltpu.VMEM((CH,), jnp.int32)] * (2 * nlists)
          + [
            pltpu.VMEM((CH,), jnp.float32),
            pltpu.VMEM((SHARE // 4,), jnp.float32),
            pltpu.VMEM_SHARED((SLICE + SINK_PAD,), jnp.float32),
            pltpu.SemaphoreType.DMA,
        ],
    )
    def sc_counts(*args):
        idx_hbm = args[:2 * nlists]
        out = args[2 * nlists]
        tbuf, vbuf = args[2 * nlists + 1:2 * nlists + 3]
        lbufs = args[2 * nlists + 3:2 * nlists + 3 + nlists]
        ibufs0 = args[2 * nlists + 3 + nlists:2 * nlists + 3 + 2 * nlists]
        ibufs1 = args[2 * nlists + 3 + 2 * nlists:2 * nlists + 3 + 3 * nlists]
        ones, zeros, acc, osem = args[2 * nlists + 3 + 3 * nlists:]

        c = lax.axis_index("c")
        s = lax.axis_index("s")
        base = jnp.minimum(s * CH, E - CH)
        lane = lax.iota(jnp.int32, L)

        def build(t_hbm, v_hbm, lbuf):
            pltpu.sync_copy(t_hbm.at[pl.ds(base, CH)], tbuf)
            pltpu.sync_copy(v_hbm.at[pl.ds(base, CH)], vbuf)

            def body(j, _):
                t = tbuf[pl.ds(j * L, L)]
                v = vbuf[pl.ds(j * L, L)]
                flat = t * N_VID + v
                gpos = base + j * L + lane
                mine = gpos >= s * CH
                lbuf[pl.ds(j * L, L)] = jnp.where(mine, flat, HUGE)
                return 0

            lax.fori_loop(0, NVREG, body, 0)

        for i in range(nlists):
            build(idx_hbm[2 * i], idx_hbm[2 * i + 1], lbufs[i])

        def fill(j, _):
            ones[pl.ds(j * L, L)] = jnp.full((L,), 1.0, jnp.float32)
            return 0

        lax.fori_loop(0, NVREG, fill, 0)

        def fillz(j, _):
            zeros[pl.ds(j * L, L)] = jnp.zeros((L,), jnp.float32)
            return 0

        lax.fori_loop(0, SHARE // 4 // L, fillz, 0)

        def zinit(k, _):
            pltpu.sync_copy(zeros, acc.at[pl.ds(s * SHARE + k * (SHARE // 4),
                                                SHARE // 4)])
            return 0

        lax.fori_loop(0, 4, zinit, 0)
        plsc.subcore_barrier()

        def mk_idx(lbuf, ibuf, lo):
            def body(j, _):
                u = lbuf[pl.ds(j * L, L)] - lo
                inr = (u >= 0) & (u < SLICE)
                sink = SLICE + ((j & 7) * L) + lane
                ibuf[pl.ds(j * L, L)] = jnp.where(inr, u, sink)
                return 0

            lax.fori_loop(0, NVREG, body, 0)

        def lo_of(p):
            return (c * ROWS_PER_SC + min(p * R, ROWS_PER_SC - R)) * N_VID

        sets = (ibufs0, ibufs1)
        off = s * SHARE

        for lbuf, ibuf in zip(lbufs, sets[0]):
            mk_idx(lbuf, ibuf, lo_of(0))
        for p in range(NPASS):
            cur = sets[p % 2]
            for ibuf in cur:
                pltpu.sync_copy(ones, acc.at[ibuf], add=True)
            plsc.subcore_barrier()

            cp = pltpu.async_copy(acc.at[pl.ds(off, SHARE)],
                                  out.at[pl.ds(lo_of(p) + off, SHARE)],
                                  osem)
            if p < NPASS - 1:
                for lbuf, ibuf in zip(lbufs, sets[(p + 1) % 2]):
                    mk_idx(lbuf, ibuf, lo_of(p + 1))
            cp.wait()
            plsc.subcore_barrier()

            if p < NPASS - 1:
                for ibuf in cur:
                    pltpu.sync_copy(zeros.at[pl.ds(0, CH)], acc.at[ibuf])
                plsc.subcore_barrier()

    return sc_counts


_sc_counts_all = _make_sc_counts(2)
_sc_counts_pos = _make_sc_counts(1)


BT = 200
BV = 2048


def _tc_body(a_ref, b_ref, cnt_ref, o_ref):
    s = lax.dot_general(
        a_ref[...], b_ref[...],
        dimension_numbers=(((1,), (1,)), ((), ())),
        preferred_element_type=jnp.float32,
    )
    o_ref[...] = s * cnt_ref[...]


def _tc_score(h_tag, h_vid, counts):
    return pl.pallas_call(
        _tc_body,
        grid=(N_TAG // BT, N_VID // BV),
        in_specs=[
            pl.BlockSpec((BT, D), lambda i, j: (i, 0)),
            pl.BlockSpec((BV, D), lambda i, j: (j, 0)),
            pl.BlockSpec((BT, BV), lambda i, j: (i, j)),
        ],
        out_specs=pl.BlockSpec((BT, BV), lambda i, j: (i, j)),
        out_shape=jax.ShapeDtypeStruct((N_TAG, N_VID), jnp.float32),
    )(h_tag, h_vid, counts)


def kernel(h_tag, h_vid, pos_tag_idx, pos_vid_idx, neg_tag_idx, neg_vid_idx):
    counts_all = _sc_counts_all(
        pos_tag_idx, pos_vid_idx, neg_tag_idx, neg_vid_idx
    ).reshape(N_TAG, N_VID)
    labels = _sc_counts_pos(pos_tag_idx, pos_vid_idx).reshape(N_TAG, N_VID)
    cls_score = _tc_score(h_tag.astype(jnp.bfloat16),
                          h_vid.astype(jnp.bfloat16), counts_all)
    return (cls_score, labels)


# speedup vs baseline: 1.3588x; 1.0043x over previous
import functools

import jax
import jax.numpy as jnp
from jax import lax
from jax.experimental import pallas as pl
from jax.experimental.pallas import tpu as pltpu
from jax.experimental.pallas import tpu_sc as plsc

N_TAG = 1000
N_VID = 16384
D = 768
E = 50000
FLAT = N_TAG * N_VID

NC = 2
NS = 16
L = 16

CH = 3136
NVREG = CH // L
ROWS_PER_SC = N_TAG // NC
R = 77
NPASS = -(-ROWS_PER_SC // R)
SLICE = R * N_VID
SHARE = SLICE // NS
SINK_PAD = 128
HUGE = 0x7FFFFFF0

_mesh = plsc.VectorSubcoreMesh(
    core_axis_name="c", subcore_axis_name="s", num_cores=NC, num_subcores=NS
)


def _make_sc_counts(nlists):
    @functools.partial(
        pl.kernel,
        out_type=jax.ShapeDtypeStruct((FLAT,), jnp.float32),
        mesh=_mesh,
        scratch_types=[
            pltpu.VMEM((CH,), jnp.int32),
            pltpu.VMEM((CH,), jnp.int32),
        ] + [pltpu.VMEM((CH,), jnp.int32)] * nlists
          + [pltpu.VMEM((nlists * CH,), jnp.int32)] * 2
          + [
            pltpu.VMEM((nlists * CH,), jnp.float32),
            pltpu.VMEM((SHARE // 4,), jnp.float32),
            pltpu.VMEM_SHARED((SLICE + SINK_PAD,), jnp.float32),
            pltpu.SemaphoreType.DMA,
        ],
    )
    def sc_counts(*args):
        idx_hbm = args[:2 * nlists]
        out = args[2 * nlists]
        tbuf, vbuf = args[2 * nlists + 1:2 * nlists + 3]
        lbufs = args[2 * nlists + 3:2 * nlists + 3 + nlists]
        ibuf0, ibuf1 = args[2 * nlists + 3 + nlists:2 * nlists + 5 + nlists]
        ones, zeros, acc, osem = args[2 * nlists + 5 + nlists:]

        c = lax.axis_index("c")
        s = lax.axis_index("s")
        base = jnp.minimum(s * CH, E - CH)
        lane = lax.iota(jnp.int32, L)

        def build(t_hbm, v_hbm, lbuf):
            pltpu.sync_copy(t_hbm.at[pl.ds(base, CH)], tbuf)
            pltpu.sync_copy(v_hbm.at[pl.ds(base, CH)], vbuf)

            def body(j, _):
                t = tbuf[pl.ds(j * L, L)]
                v = vbuf[pl.ds(j * L, L)]
                flat = t * N_VID + v
                gpos = base + j * L + lane
                mine = gpos >= s * CH
                lbuf[pl.ds(j * L, L)] = jnp.where(mine, flat, HUGE)
                return 0

            lax.fori_loop(0, NVREG, body, 0)

        for i in range(nlists):
            build(idx_hbm[2 * i], idx_hbm[2 * i + 1], lbufs[i])

        def fill(j, _):
            ones[pl.ds(j * L, L)] = jnp.full((L,), 1.0, jnp.float32)
            return 0

        lax.fori_loop(0, nlists * NVREG, fill, 0)

        def fillz(j, _):
            zeros[pl.ds(j * L, L)] = jnp.zeros((L,), jnp.float32)
            return 0

        lax.fori_loop(0, SHARE // 4 // L, fillz, 0)

        def zinit(k, _):
            pltpu.sync_copy(zeros, acc.at[pl.ds(s * SHARE + k * (SHARE // 4),
                                                SHARE // 4)])
            return 0

        lax.fori_loop(0, 4, zinit, 0)
        plsc.subcore_barrier()

        def mk_idx(lbuf, ibuf, dst, lo):
            def body(j, _):
                u = lbuf[pl.ds(j * L, L)] - lo
                inr = (u >= 0) & (u < SLICE)
                sink = SLICE + ((j & 7) * L) + lane
                ibuf[pl.ds(dst + j * L, L)] = jnp.where(inr, u, sink)
                return 0

            lax.fori_loop(0, NVREG, body, 0)

        def mk_all(ibuf, lo):
            for i, lbuf in enumerate(lbufs):
                mk_idx(lbuf, ibuf, i * CH, lo)

        def lo_of(p):
            return (c * ROWS_PER_SC + min(p * R, ROWS_PER_SC - R)) * N_VID

        sets = (ibuf0, ibuf1)
        off = s * SHARE

        mk_all(sets[0], lo_of(0))
        for p in range(NPASS):
            cur = sets[p % 2]
            pltpu.sync_copy(ones, acc.at[cur], add=True)
            plsc.subcore_barrier()

            cp = pltpu.async_copy(acc.at[pl.ds(off, SHARE)],
                                  out.at[pl.ds(lo_of(p) + off, SHARE)],
                                  osem)
            if p < NPASS - 1:
                mk_all(sets[(p + 1) % 2], lo_of(p + 1))
            cp.wait()
            plsc.subcore_barrier()

            if p < NPASS - 1:
                pltpu.sync_copy(zeros.at[pl.ds(0, nlists * CH)], acc.at[cur])
                plsc.subcore_barrier()

    return sc_counts


_sc_counts_all = _make_sc_counts(2)
_sc_counts_pos = _make_sc_counts(1)


BT = 200
BV = 2048


def _tc_body(a_ref, b_ref, cnt_ref, o_ref):
    s = lax.dot_general(
        a_ref[...], b_ref[...],
        dimension_numbers=(((1,), (1,)), ((), ())),
        preferred_element_type=jnp.float32,
    )
    o_ref[...] = s * cnt_ref[...]


def _tc_score(h_tag, h_vid, counts):
    return pl.pallas_call(
        _tc_body,
        grid=(N_TAG // BT, N_VID // BV),
        in_specs=[
            pl.BlockSpec((BT, D), lambda i, j: (i, 0)),
            pl.BlockSpec((BV, D), lambda i, j: (j, 0)),
            pl.BlockSpec((BT, BV), lambda i, j: (i, j)),
        ],
        out_specs=pl.BlockSpec((BT, BV), lambda i, j: (i, j)),
        out_shape=jax.ShapeDtypeStruct((N_TAG, N_VID), jnp.float32),
    )(h_tag, h_vid, counts)


def kernel(h_tag, h_vid, pos_tag_idx, pos_vid_idx, neg_tag_idx, neg_vid_idx):
    counts_all = _sc_counts_all(
        pos_tag_idx, pos_vid_idx, neg_tag_idx, neg_vid_idx
    ).reshape(N_TAG, N_VID)
    labels = _sc_counts_pos(pos_tag_idx, pos_vid_idx).reshape(N_TAG, N_VID)
    cls_score = _tc_score(h_tag.astype(jnp.bfloat16),
                          h_vid.astype(jnp.bfloat16), counts_all)
    return (cls_score, labels)
